# baseline (speedup 1.0000x reference)
import jax
import jax.numpy as jnp
from jax.experimental import pallas as pl

_TOKENS_PER_TILE = 256


def _moe_lora_tile(x_ref, wrt_ref, a2_ref, b2_ref, out_ref):
    x = x_ref[...]
    scores = jnp.dot(x, wrt_ref[...],
                     preferred_element_type=jnp.float32)
    t, k = scores.shape
    kio = jax.lax.broadcasted_iota(jnp.int32, (t, k), 1)
    m1 = jnp.max(scores, axis=1, keepdims=True)
    i1 = jnp.min(jnp.where(scores == m1, kio, k), axis=1, keepdims=True)
    s2 = jnp.where(kio == i1, -jnp.inf, scores)
    m2 = jnp.max(s2, axis=1, keepdims=True)
    i2 = jnp.min(jnp.where(s2 == m2, kio, k), axis=1, keepdims=True)
    g1 = 1.0 / (1.0 + jnp.exp(m2 - m1))
    g2 = 1.0 - g1

    h = jnp.dot(x, a2_ref[...], preferred_element_type=jnp.float32)
    kr = h.shape[1]
    r = kr // k
    eio = jax.lax.broadcasted_iota(jnp.int32, (t, kr), 1) // r
    mask = (jnp.where(eio == i1, g1, 0.0)
            + jnp.where(eio == i2, g2, 0.0))
    out_ref[...] = jnp.dot(h * mask, b2_ref[...],
                           preferred_element_type=jnp.float32)


def kernel(x, A, Bmat, Wr, alpha_over_r):
    b, s, d = x.shape
    k, r, _ = A.shape
    kr = k * r
    n_tok = b * s
    tile = _TOKENS_PER_TILE

    x2 = x.reshape(n_tok, d)
    wrt = Wr.T
    a2 = A.reshape(kr, d).T
    b2 = (Bmat.transpose(0, 2, 1).reshape(kr, d)
          * jnp.asarray(alpha_over_r, x.dtype))

    out = pl.pallas_call(
        _moe_lora_tile,
        grid=(n_tok // tile,),
        in_specs=[
            pl.BlockSpec((tile, d), lambda i: (i, 0)),
            pl.BlockSpec((d, k), lambda i: (0, 0)),
            pl.BlockSpec((d, kr), lambda i: (0, 0)),
            pl.BlockSpec((kr, d), lambda i: (0, 0)),
        ],
        out_specs=pl.BlockSpec((tile, d), lambda i: (i, 0)),
        out_shape=jax.ShapeDtypeStruct((n_tok, d), x.dtype),
    )(x2, wrt, a2, b2)
    return out.reshape(b, s, d)


# speedup vs baseline: 1.5411x; 1.5411x over previous
import jax
import jax.numpy as jnp
from jax.experimental import pallas as pl

_TOKENS_PER_TILE = 1024


def _moe_lora_tile(x_ref, wrt_ref, a2_ref, b2_ref, out_ref):
    x = x_ref[...]
    scores = jnp.dot(x, wrt_ref[...],
                     preferred_element_type=jnp.float32)
    t, k = scores.shape
    kio = jax.lax.broadcasted_iota(jnp.int32, (t, k), 1)
    m1 = jnp.max(scores, axis=1, keepdims=True)
    i1 = jnp.min(jnp.where(scores == m1, kio, k), axis=1, keepdims=True)
    s2 = jnp.where(kio == i1, -jnp.inf, scores)
    m2 = jnp.max(s2, axis=1, keepdims=True)
    i2 = jnp.min(jnp.where(s2 == m2, kio, k), axis=1, keepdims=True)
    g1 = 1.0 / (1.0 + jnp.exp(m2 - m1))
    g2 = 1.0 - g1

    h = jnp.dot(x, a2_ref[...], preferred_element_type=jnp.float32)
    kr = h.shape[1]
    r = kr // k
    eio = jax.lax.broadcasted_iota(jnp.int32, (t, kr), 1) // r
    mask = (jnp.where(eio == i1, g1, 0.0)
            + jnp.where(eio == i2, g2, 0.0))
    out_ref[...] = jnp.dot(h * mask, b2_ref[...],
                           preferred_element_type=jnp.float32)


def kernel(x, A, Bmat, Wr, alpha_over_r):
    b, s, d = x.shape
    k, r, _ = A.shape
    kr = k * r
    n_tok = b * s
    tile = _TOKENS_PER_TILE

    x2 = x.reshape(n_tok, d)
    wrt = Wr.T
    a2 = A.reshape(kr, d).T
    b2 = (Bmat.transpose(0, 2, 1).reshape(kr, d)
          * jnp.asarray(alpha_over_r, x.dtype))

    out = pl.pallas_call(
        _moe_lora_tile,
        grid=(n_tok // tile,),
        in_specs=[
            pl.BlockSpec((tile, d), lambda i: (i, 0)),
            pl.BlockSpec((d, k), lambda i: (0, 0)),
            pl.BlockSpec((d, kr), lambda i: (0, 0)),
            pl.BlockSpec((kr, d), lambda i: (0, 0)),
        ],
        out_specs=pl.BlockSpec((tile, d), lambda i: (i, 0)),
        out_shape=jax.ShapeDtypeStruct((n_tok, d), x.dtype),
    )(x2, wrt, a2, b2)
    return out.reshape(b, s, d)


# speedup vs baseline: 1.6025x; 1.0398x over previous
import jax
import jax.numpy as jnp
from jax.experimental import pallas as pl

_TOKENS_PER_TILE = 2048


def _moe_lora_tile(x_ref, wrt_ref, a2_ref, b2_ref, out_ref):
    x = x_ref[...]
    scores = jnp.dot(x, wrt_ref[...],
                     preferred_element_type=jnp.float32)
    t, k = scores.shape
    kio = jax.lax.broadcasted_iota(jnp.int32, (t, k), 1)
    m1 = jnp.max(scores, axis=1, keepdims=True)
    i1 = jnp.min(jnp.where(scores == m1, kio, k), axis=1, keepdims=True)
    s2 = jnp.where(kio == i1, -jnp.inf, scores)
    m2 = jnp.max(s2, axis=1, keepdims=True)
    i2 = jnp.min(jnp.where(s2 == m2, kio, k), axis=1, keepdims=True)
    g1 = 1.0 / (1.0 + jnp.exp(m2 - m1))
    g2 = 1.0 - g1

    h = jnp.dot(x, a2_ref[...], preferred_element_type=jnp.float32)
    kr = h.shape[1]
    r = kr // k
    eio = jax.lax.broadcasted_iota(jnp.int32, (t, kr), 1) // r
    mask = (jnp.where(eio == i1, g1, 0.0)
            + jnp.where(eio == i2, g2, 0.0))
    out_ref[...] = jnp.dot(h * mask, b2_ref[...],
                           preferred_element_type=jnp.float32)


def kernel(x, A, Bmat, Wr, alpha_over_r):
    b, s, d = x.shape
    k, r, _ = A.shape
    kr = k * r
    n_tok = b * s
    tile = _TOKENS_PER_TILE

    x2 = x.reshape(n_tok, d)
    wrt = Wr.T
    a2 = A.reshape(kr, d).T
    b2 = (Bmat.transpose(0, 2, 1).reshape(kr, d)
          * jnp.asarray(alpha_over_r, x.dtype))

    out = pl.pallas_call(
        _moe_lora_tile,
        grid=(n_tok // tile,),
        in_specs=[
            pl.BlockSpec((tile, d), lambda i: (i, 0)),
            pl.BlockSpec((d, k), lambda i: (0, 0)),
            pl.BlockSpec((d, kr), lambda i: (0, 0)),
            pl.BlockSpec((kr, d), lambda i: (0, 0)),
        ],
        out_specs=pl.BlockSpec((tile, d), lambda i: (i, 0)),
        out_shape=jax.ShapeDtypeStruct((n_tok, d), x.dtype),
    )(x2, wrt, a2, b2)
    return out.reshape(b, s, d)


# speedup vs baseline: 1.6184x; 1.0099x over previous
import jax
import jax.numpy as jnp
from jax.experimental import pallas as pl

_TOKENS_PER_TILE = 2048


def _moe_lora_tile(x_ref, wrt_ref, a2_ref, b2_ref, out_ref):
    x = x_ref[...]
    scores = jnp.dot(x, wrt_ref[...],
                     preferred_element_type=jnp.float32)
    t, k = scores.shape
    kio = jax.lax.broadcasted_iota(jnp.int32, (t, k), 1)
    m1 = jnp.max(scores, axis=1, keepdims=True)
    i1 = jnp.min(jnp.where(scores == m1, kio, k), axis=1, keepdims=True)
    s2 = jnp.where(kio == i1, -jnp.inf, scores)
    m2 = jnp.max(s2, axis=1, keepdims=True)
    i2 = jnp.min(jnp.where(s2 == m2, kio, k), axis=1, keepdims=True)
    g1 = 1.0 / (1.0 + jnp.exp(m2 - m1))
    g2 = 1.0 - g1

    h = jnp.dot(x.astype(jnp.bfloat16), a2_ref[...],
                preferred_element_type=jnp.float32)
    kr = h.shape[1]
    r = kr // k
    eio = jax.lax.broadcasted_iota(jnp.int32, (t, kr), 1) // r
    mask = (jnp.where(eio == i1, g1, 0.0)
            + jnp.where(eio == i2, g2, 0.0))
    out_ref[...] = jnp.dot((h * mask).astype(jnp.bfloat16), b2_ref[...],
                           preferred_element_type=jnp.float32)


def kernel(x, A, Bmat, Wr, alpha_over_r):
    b, s, d = x.shape
    k, r, _ = A.shape
    kr = k * r
    n_tok = b * s
    tile = _TOKENS_PER_TILE

    x2 = x.reshape(n_tok, d)
    wrt = Wr.T
    a2 = A.reshape(kr, d).T.astype(jnp.bfloat16)
    b2 = (Bmat.transpose(0, 2, 1).reshape(kr, d)
          * jnp.asarray(alpha_over_r, x.dtype)).astype(jnp.bfloat16)

    out = pl.pallas_call(
        _moe_lora_tile,
        grid=(n_tok // tile,),
        in_specs=[
            pl.BlockSpec((tile, d), lambda i: (i, 0)),
            pl.BlockSpec((d, k), lambda i: (0, 0)),
            pl.BlockSpec((d, kr), lambda i: (0, 0)),
            pl.BlockSpec((kr, d), lambda i: (0, 0)),
        ],
        out_specs=pl.BlockSpec((tile, d), lambda i: (i, 0)),
        out_shape=jax.ShapeDtypeStruct((n_tok, d), x.dtype),
    )(x2, wrt, a2, b2)
    return out.reshape(b, s, d)


# speedup vs baseline: 1.6984x; 1.0494x over previous
import jax
import jax.numpy as jnp
from jax.experimental import pallas as pl

_TOKENS_PER_TILE = 2048


def _moe_lora_tile(x_ref, wrt_ref, a2_ref, b2_ref, out_ref):
    x = x_ref[...]
    scores = jnp.dot(x, wrt_ref[...],
                     preferred_element_type=jnp.float32)
    m1 = jnp.max(scores, axis=1, keepdims=True)
    is1 = scores == m1
    s2 = jnp.where(is1, -jnp.inf, scores)
    m2 = jnp.max(s2, axis=1, keepdims=True)
    g1 = 1.0 / (1.0 + jnp.exp(m2 - m1))
    g2 = 1.0 - g1
    w = jnp.where(is1, g1, 0.0) + jnp.where(s2 == m2, g2, 0.0)

    h = jnp.dot(x.astype(jnp.bfloat16), a2_ref[...],
                preferred_element_type=jnp.float32)
    out_ref[...] = jnp.dot((h * w).astype(jnp.bfloat16), b2_ref[...],
                           preferred_element_type=jnp.float32)


def kernel(x, A, Bmat, Wr, alpha_over_r):
    b, s, d = x.shape
    k, r, _ = A.shape
    kr = k * r
    n_tok = b * s
    tile = _TOKENS_PER_TILE

    x2 = x.reshape(n_tok, d)
    wrt = jnp.repeat(Wr, r, axis=0).T
    a2 = A.reshape(kr, d).T.astype(jnp.bfloat16)
    b2 = (Bmat.transpose(0, 2, 1).reshape(kr, d)
          * jnp.asarray(alpha_over_r, x.dtype)).astype(jnp.bfloat16)

    out = pl.pallas_call(
        _moe_lora_tile,
        grid=(n_tok // tile,),
        in_specs=[
            pl.BlockSpec((tile, d), lambda i: (i, 0)),
            pl.BlockSpec((d, kr), lambda i: (0, 0)),
            pl.BlockSpec((d, kr), lambda i: (0, 0)),
            pl.BlockSpec((kr, d), lambda i: (0, 0)),
        ],
        out_specs=pl.BlockSpec((tile, d), lambda i: (i, 0)),
        out_shape=jax.ShapeDtypeStruct((n_tok, d), x.dtype),
    )(x2, wrt, a2, b2)
    return out.reshape(b, s, d)
